# baseline (speedup 1.0000x reference)
import functools

import numpy as np
import jax
import jax.numpy as jnp
from jax import lax
from jax.experimental import pallas as pl
from jax.experimental.pallas import tpu as pltpu
from jax.experimental.pallas import tpu_sc as plsc

B = 1024
NJ = 14
COL = 14
NW = 32
BPW = B // NW


def _build_tables():
    radius = 4
    xk = np.arange(-radius, radius + 1)
    k = np.exp(-0.5 * xk.astype(np.float64) ** 2)
    k = (k / k.sum()).astype(np.float32)
    prof = np.zeros((COL, COL), np.float32)
    for c in range(COL):
        d = np.zeros(COL, np.float32)
        d[c] = 1.0
        p = np.pad(d, radius, mode='symmetric')
        for i in range(COL):
            prof[c, i] = np.dot(k, p[i:i + 2 * radius + 1])
    T = prof / prof.max(axis=1, keepdims=True)
    S2 = (T * T).sum(axis=1)
    return T, S2


_T_np, _S2_np = _build_tables()


def _sc_loss(os, h, tv, t14, s2):
    mesh = plsc.VectorSubcoreMesh(core_axis_name="c", subcore_axis_name="s")

    @functools.partial(
        pl.kernel, mesh=mesh,
        compiler_params=pltpu.CompilerParams(needs_layout_passes=False),
        out_type=jax.ShapeDtypeStruct((NW, 2, 16), jnp.float32),
        scratch_types=[
            pltpu.VMEM((NJ * COL, COL), jnp.float32),
            pltpu.VMEM((2 * NJ * COL, COL), jnp.float32),
            pltpu.VMEM((BPW, 4 * NJ), jnp.float32),
            pltpu.VMEM((COL, COL), jnp.float32),
            pltpu.VMEM((COL,), jnp.float32),
            pltpu.VMEM((2, 16), jnp.float32),
            pltpu.SemaphoreType.DMA,
        ],
    )
    def k(os_hbm, h_hbm, tv_hbm, t14_hbm, s2_hbm, out_hbm,
          h_s, os_s, tv_s, t14_s, s2_s, out_s, sem):
        wid = lax.axis_index("s") * 2 + lax.axis_index("c")
        b0 = wid * BPW
        pltpu.sync_copy(t14_hbm, t14_s)
        pltpu.sync_copy(s2_hbm, s2_s)
        pltpu.sync_copy(tv_hbm.at[pl.ds(b0, BPW)], tv_s)

        z16 = lax.iota(jnp.int32, 16)
        jv = jnp.minimum(z16, NJ - 1)
        lmaskf = (z16 < NJ).astype(jnp.float32)
        zero = jnp.zeros((16,), jnp.int32)
        one = zero + 1
        fz = jnp.zeros((16,), jnp.float32)
        scale = 1.0 / COL

        def body_b(bl, carry):
            acc_main, acc2v, accvs = carry
            b = b0 + bl
            cps = []
            for j in range(NJ):
                cps.append(pltpu.async_copy(
                    h_hbm.at[b, j], h_s.at[pl.ds(j * COL, COL)], sem))
            for ch in range(2 * NJ):
                cps.append(pltpu.async_copy(
                    os_hbm.at[b, ch], os_s.at[pl.ds(ch * COL, COL)], sem))

            blv = zero + bl
            c4 = jv * 4
            txg = plsc.load_gather(tv_s, [blv, c4])
            tyg = plsc.load_gather(tv_s, [blv, c4 + 1])
            v0g = plsc.load_gather(tv_s, [blv, c4 + 2]) * lmaskf
            v1g = plsc.load_gather(tv_s, [blv, c4 + 3]) * lmaskf
            xi = jnp.clip((txg * COL).astype(jnp.int32), 0, COL - 1)
            yi = jnp.clip((tyg * COL).astype(jnp.int32), 0, COL - 1)

            for cp in cps:
                cp.wait()

            def body_y(y, cy):
                best, bestl, sq, bil = cy
                yv = zero + y
                tyw = plsc.load_gather(t14_s, [yi, yv])
                rh = jv * COL + yv

                def body_x(x, cx):
                    best, bestl, sq, bil = cx
                    xv = zero + x
                    hv = plsc.load_gather(h_s, [rh, xv])
                    hvm = hv * lmaskf
                    sq = sq + hvm * hvm
                    txw = plsc.load_gather(t14_s, [xi, xv])
                    bil = bil + hvm * (tyw * txw)
                    gt = hv > best
                    lv = zero + (y * COL + x)
                    best = jnp.where(gt, hv, best)
                    bestl = jnp.where(gt, lv, bestl)
                    return best, bestl, sq, bil

                return lax.fori_loop(0, COL, body_x, (best, bestl, sq, bil))

            best, bestl, sq, bil = lax.fori_loop(
                0, COL, body_y,
                (fz - 1.0, zero, fz, fz))

            jc0 = jv * COL

            def body_r0(x, r0):
                hv = plsc.load_gather(h_s, [jc0, zero + x]) * lmaskf
                return r0 + hv * hv

            r0acc = lax.fori_loop(0, COL, body_r0, fz)

            vis = v0g == 1.0
            c2 = jnp.where(vis, -2.0, 0.0)
            tts = plsc.load_gather(s2_s, [yi]) * plsc.load_gather(s2_s, [xi])
            acc_main = (acc_main + sq + c2 * bil
                        + jnp.where(vis, tts, 0.0)
                        - jnp.where(vis, 0.0, r0acc))

            yC = bestl // COL
            xC = bestl - yC * COL
            maskv = best > 0.5
            gx = jnp.where(maskv, v0g * scale, 0.0)
            ux = v0g * jnp.where(maskv, xC.astype(jnp.float32) * scale - txg,
                                 -txg)
            gy = jnp.where(maskv, v1g * scale, 0.0)
            uy = v1g * jnp.where(maskv, yC.astype(jnp.float32) * scale - tyg,
                                 -tyg)

            ox = plsc.load_gather(os_s, [jv * COL + yC, xC])
            oy = plsc.load_gather(os_s, [(jv + NJ) * COL + yC, xC])
            dx = gx * ox + ux
            dy = gy * oy + uy
            acc2v = acc2v + dx * dx + dy * dy
            accvs = accvs + v0g + v1g
            return acc_main, acc2v, accvs

        acc_main, acc2v, accvs = lax.fori_loop(
            0, BPW, body_b, (fz, fz, fz))

        out_s[0, :] = acc_main + acc2v
        out_s[1, :] = accvs
        pltpu.sync_copy(out_s, out_hbm.at[wid])

    return k(os, h, tv, t14, s2)


@jax.jit
def _run(os, h, t, v):
    t14 = jnp.asarray(_T_np)
    s2 = jnp.asarray(_S2_np)
    tv = jnp.concatenate([t, v], axis=-1).reshape(B, 4 * NJ)
    partials = _sc_loss(os, h, tv, t14, s2)
    sums = jnp.sum(partials, axis=(0, 2))
    return sums[0] / (sums[1] * 0.5)


def kernel(os, h, op, t, v):
    return _run(os, h, t, v)
